# Initial kernel scaffold; baseline (speedup 1.0000x reference)
import jax
import jax.numpy as jnp
from jax.experimental import pallas as pl


def kernel(p1, x1, p2, x2, W1, b1, g1, be1, W2, b2, g2, be2):
    raise NotImplementedError("write your pallas kernel here")


# speedup vs baseline: 45.4256x; 45.4256x over previous
import functools

import jax
import jax.numpy as jnp
from jax import lax
from jax.experimental import pallas as pl

_BIG = 1e30


def _linear_stats_body(x_ref, wt_ref, b_ref, y_ref, acc_ref):
    y = jnp.dot(x_ref[...], wt_ref[...],
                preferred_element_type=jnp.float32) + b_ref[...]
    y_ref[...] = y

    @pl.when(pl.program_id(0) == 0)
    def _():
        acc_ref[...] = jnp.zeros_like(acc_ref)

    s = jnp.sum(y, axis=0, keepdims=True)
    ss = jnp.sum(y * y, axis=0, keepdims=True)
    acc_ref[...] += jnp.concatenate([s, ss], axis=0)


def _linear_stats(x, wt, b, tile):
    n, _ = x.shape
    c = wt.shape[1]
    grid = (n // tile,)
    return pl.pallas_call(
        _linear_stats_body,
        grid=grid,
        in_specs=[
            pl.BlockSpec((tile, x.shape[1]), lambda i: (i, 0)),
            pl.BlockSpec(wt.shape, lambda i: (0, 0)),
            pl.BlockSpec((1, c), lambda i: (0, 0)),
        ],
        out_specs=[
            pl.BlockSpec((tile, c), lambda i: (i, 0)),
            pl.BlockSpec((2, c), lambda i: (0, 0)),
        ],
        out_shape=[
            jax.ShapeDtypeStruct((n, c), jnp.float32),
            jax.ShapeDtypeStruct((2, c), jnp.float32),
        ],
    )(x, wt, b)


def _affine_relu_body(y_ref, sc_ref, sh_ref, z_ref):
    z_ref[...] = jnp.maximum(y_ref[...] * sc_ref[...] + sh_ref[...], 0.0)


def _affine_relu(y, sc, sh, tile):
    n, c = y.shape
    return pl.pallas_call(
        _affine_relu_body,
        grid=(n // tile,),
        in_specs=[
            pl.BlockSpec((tile, c), lambda i: (i, 0)),
            pl.BlockSpec((1, c), lambda i: (0, 0)),
            pl.BlockSpec((1, c), lambda i: (0, 0)),
        ],
        out_specs=pl.BlockSpec((tile, c), lambda i: (i, 0)),
        out_shape=jax.ShapeDtypeStruct((n, c), jnp.float32),
    )(y, sc, sh)


def _knn_interp_body(p1_ref, p2t_ref, z2_ref, y1_ref, sc_ref, sh_ref, out_ref):
    a = p1_ref[0]
    pt = p2t_ref[0]
    acc = jnp.zeros((a.shape[0], pt.shape[1]), jnp.float32)
    for d in range(3):
        t = a[:, d:d + 1] - pt[d:d + 1, :]
        acc = acc + t * t

    m1 = jnp.min(acc, axis=1, keepdims=True)
    eq1 = acc == m1
    acc1 = jnp.where(eq1, _BIG, acc)
    m2 = jnp.min(acc1, axis=1, keepdims=True)
    eq2 = acc1 == m2
    acc2 = jnp.where(eq2, _BIG, acc1)
    m3 = jnp.min(acc2, axis=1, keepdims=True)
    eq3 = acc2 == m3

    r1 = 1.0 / (m1 + 1e-8)
    r2 = 1.0 / (m2 + 1e-8)
    r3 = 1.0 / (m3 + 1e-8)
    inv = 1.0 / (r1 + r2 + r3)
    zero = jnp.float32(0.0)
    blend = (jnp.where(eq1, r1 * inv, zero)
             + jnp.where(eq2, r2 * inv, zero)
             + jnp.where(eq3, r3 * inv, zero))

    interp = jnp.dot(blend, z2_ref[0], preferred_element_type=jnp.float32)
    out_ref[0] = interp + jnp.maximum(
        y1_ref[0] * sc_ref[...] + sh_ref[...], 0.0)


def _knn_interp(p1, p2t, z2, y1, sc1, sh1, tile):
    b, n1, _ = p1.shape
    n2 = p2t.shape[2]
    c = z2.shape[2]
    grid = (b, n1 // tile)
    return pl.pallas_call(
        _knn_interp_body,
        grid=grid,
        in_specs=[
            pl.BlockSpec((1, tile, 3), lambda i, j: (i, j, 0)),
            pl.BlockSpec((1, 3, n2), lambda i, j: (i, 0, 0)),
            pl.BlockSpec((1, n2, c), lambda i, j: (i, 0, 0)),
            pl.BlockSpec((1, tile, c), lambda i, j: (i, j, 0)),
            pl.BlockSpec((1, c), lambda i, j: (0, 0)),
            pl.BlockSpec((1, c), lambda i, j: (0, 0)),
        ],
        out_specs=pl.BlockSpec((1, tile, c), lambda i, j: (i, j, 0)),
        out_shape=jax.ShapeDtypeStruct((b, n1, c), jnp.float32),
    )(p1, p2t, z2, y1, sc1, sh1)


def _bn_affine(acc, n, gamma, beta, eps=1e-5):
    mean = acc[0] / n
    var = acc[1] / n - mean * mean
    sc = gamma * lax.rsqrt(var + eps)
    sh = beta - mean * sc
    return sc[None, :], sh[None, :]


@jax.jit
def kernel(p1, x1, p2, x2, W1, b1, g1, be1, W2, b2, g2, be2):
    B, N1, _ = p1.shape
    N2 = p2.shape[1]
    C = W1.shape[0]

    y2, acc2 = _linear_stats(x2.reshape(B * N2, -1), W2.T, b2[None, :], 1024)
    sc2, sh2 = _bn_affine(acc2, B * N2, g2, be2)
    z2 = _affine_relu(y2, sc2, sh2, 1024).reshape(B, N2, C)

    y1, acc1 = _linear_stats(x1.reshape(B * N1, -1), W1.T, b1[None, :], 1024)
    sc1, sh1 = _bn_affine(acc1, B * N1, g1, be1)

    p2t = jnp.transpose(p2, (0, 2, 1))
    out = _knn_interp(p1, p2t, z2, y1.reshape(B, N1, C), sc1, sh1, 512)
    return out
